# baseline (speedup 1.0000x reference)
import functools
import math

import jax
import jax.numpy as jnp
from jax import lax
from jax.experimental import pallas as pl
from jax.experimental.pallas import tpu as pltpu
from jax.experimental.pallas import tpu_sc as plsc

N_NODES_C = 10000
NUM_SPECIES_C = 100
NW = 32
LANES = 16

_SB = 512
_NSEG = 5
_BE = _SB * _NSEG


def _silu(v):
    t = 0.5 * v
    return t * jnp.tanh(t) + t


def _edge1_body(vec_ref, x_ref, wcat_ref, w2bdt_ref, v3t_ref, *out_refs):
    xb16 = x_ref[...].astype(jnp.bfloat16)
    h = _silu(jnp.dot(xb16, wcat_ref[...], preferred_element_type=jnp.float32))
    v = vec_ref[...]
    tr = (((1,), (1,)), ((), ()))
    st = (lax.dot_general(w2bdt_ref[...], h.astype(jnp.bfloat16), tr,
                          preferred_element_type=jnp.float32)
          + lax.dot_general(v3t_ref[...], v * v, tr,
                            preferred_element_type=jnp.float32))
    r2 = st[3]
    u = jnp.clip(jnp.sqrt(r2), 0.0, 1.0)
    u2 = u * u
    u6 = u2 * u2 * u2
    env = 1.0 - 28.0 * u6 + 48.0 * u6 * u - 21.0 * u6 * u2
    rows = (st[0], st[1], st[2], env)
    for q in range(4):
        for j in range(_NSEG):
            out_refs[q * _NSEG + j][...] = rows[q][j * _SB:(j + 1) * _SB]


def _edge_stage1(vectors, x, wcat, w2bd, v3):
    n_edges = x.shape[0]
    grid = n_edges // _BE
    seg = n_edges // _NSEG
    blk = lambda shape: pl.BlockSpec(shape, lambda i: (i, 0))
    blk1 = pl.BlockSpec((_SB,), lambda i: (i,))
    full = lambda shape: pl.BlockSpec(shape, lambda i: (0, 0))
    out_sds = jax.ShapeDtypeStruct((seg,), jnp.float32)
    return pl.pallas_call(
        _edge1_body,
        grid=(grid,),
        in_specs=[blk((_BE, 3)), blk((_BE, 128)),
                  full((128, 80)), full((8, 80)), full((8, 3))],
        out_specs=[blk1] * (4 * _NSEG),
        out_shape=[out_sds] * (4 * _NSEG),
    )(vectors, x, wcat, w2bd, v3)


def _make_seg_kernel(n_edges, n_nodes):
    e_pw = n_edges // NW
    mesh = plsc.VectorSubcoreMesh(core_axis_name="c", subcore_axis_name="s")
    out_sds = jax.ShapeDtypeStruct((NW, n_nodes), jnp.float32)

    @functools.partial(
        pl.kernel, mesh=mesh,
        out_type=[out_sds] * 3,
        compiler_params=pltpu.CompilerParams(needs_layout_passes=False),
        scratch_types=[
            pltpu.VMEM((e_pw,), jnp.float32),
            pltpu.VMEM((e_pw,), jnp.float32),
            pltpu.VMEM((e_pw,), jnp.float32),
            pltpu.VMEM((e_pw,), jnp.int32),
            pltpu.VMEM((n_nodes,), jnp.float32),
            pltpu.VMEM((n_nodes,), jnp.float32),
            pltpu.VMEM((n_nodes,), jnp.float32),
        ],
    )
    def seg_kernel(ce_hbm, se_hbm, ee_hbm, snd_hbm,
                   outc_hbm, outs_hbm, oute_hbm,
                   vc, vs, ve, vidx, ac, a_s, ae):
        wid = lax.axis_index("s") * 2 + lax.axis_index("c")
        base = wid * e_pw
        pltpu.sync_copy(ce_hbm.at[pl.ds(base, e_pw)], vc)
        pltpu.sync_copy(se_hbm.at[pl.ds(base, e_pw)], vs)
        pltpu.sync_copy(ee_hbm.at[pl.ds(base, e_pw)], ve)
        pltpu.sync_copy(snd_hbm.at[pl.ds(base, e_pw)], vidx)

        zero = jnp.zeros((LANES,), jnp.float32)

        def zero_body(i, carry):
            sl = pl.ds(i * LANES, LANES)
            ac[sl] = zero
            a_s[sl] = zero
            ae[sl] = zero
            return carry

        lax.fori_loop(0, n_nodes // LANES, zero_body, 0)

        def scat_body(i, carry):
            sl = pl.ds(i * LANES, LANES)
            idx = vidx[sl]
            plsc.addupdate_scatter(ac, [idx], vc[sl])
            plsc.addupdate_scatter(a_s, [idx], vs[sl])
            plsc.addupdate_scatter(ae, [idx], ve[sl])
            return carry

        lax.fori_loop(0, e_pw // LANES, scat_body, 0)

        pltpu.sync_copy(ac, outc_hbm.at[wid])
        pltpu.sync_copy(a_s, outs_hbm.at[wid])
        pltpu.sync_copy(ae, oute_hbm.at[wid])

    return seg_kernel


def _node_body(cp_ref, sp_ref, ep_ref, spe_ref, rad_ref, hardn_ref,
               cemb_ref, w0_ref, wrest_ref, w1b_ref,
               charges_ref, pot_ref, vdw_ref, sprime_ref, uvec_ref):
    chis = jnp.sum(cp_ref[...], axis=0, keepdims=True)
    ssum = jnp.sum(sp_ref[...], axis=0, keepdims=True)
    esum = jnp.sum(ep_ref[...], axis=0, keepdims=True)
    spe = spe_ref[...]
    n_sp = rad_ref.shape[1]
    ioc = lax.broadcasted_iota(jnp.int32, (n_sp, 1), 0)
    oh = (spe == ioc).astype(jnp.float32)
    gam = jnp.dot(rad_ref[...], oh, preferred_element_type=jnp.float32) * 4.0 + 0.5
    hraw = jnp.dot(hardn_ref[...], oh, preferred_element_type=jnp.float32)
    hard = jnp.maximum(hraw, 0.0) + jnp.log1p(jnp.exp(-jnp.abs(hraw)))
    charges = -chis / hard
    pot_terms = (chis * charges + 0.5 * hard * charges * charges
                 + charges * charges / (gam * math.sqrt(math.pi)))
    pot_ref[...] = jnp.sum(pot_terms, keepdims=True)
    sigma = jax.nn.sigmoid(ssum) * 0.15 + 0.15
    epsn = jax.nn.sigmoid(esum) * 1.7 + 0.3
    s2 = sigma * sigma
    vdw_ref[...] = jnp.sum(epsn * s2 * s2 * s2, keepdims=True)
    charges_ref[...] = charges
    s_tab = jnp.dot(cemb_ref[...], wrest_ref[...], preferred_element_type=jnp.float32)
    sprime_ref[...] = jnp.dot(s_tab, w1b_ref[...], preferred_element_type=jnp.float32)
    uvec_ref[...] = jnp.dot(w0_ref[...], w1b_ref[...], preferred_element_type=jnp.float32)


def _node_stage(cparts, sparts, eparts, species_row, rad_row, hardn_row,
                charge_embed, w0, wrest, w1b):
    n_nodes = species_row.shape[1]
    n_sp = rad_row.shape[1]
    full = lambda shape: pl.BlockSpec(shape, lambda: (0, 0))
    return pl.pallas_call(
        _node_body,
        in_specs=[full((NW, n_nodes))] * 3 + [
            full((1, n_nodes)), full((1, n_sp)), full((1, n_sp)),
            full((n_sp, 16)), full((1, 16)), full((16, 16)), full((16, 128))],
        out_specs=[full((1, n_nodes)), full((1, 1)), full((1, 1)),
                   full((n_sp, 128)), full((1, 128))],
        out_shape=[jax.ShapeDtypeStruct((1, n_nodes), jnp.float32),
                   jax.ShapeDtypeStruct((1, 1), jnp.float32),
                   jax.ShapeDtypeStruct((1, 1), jnp.float32),
                   jax.ShapeDtypeStruct((n_sp, 128), jnp.float32),
                   jax.ShapeDtypeStruct((1, 128), jnp.float32)],
    )(cparts, sparts, eparts, species_row, rad_row, hardn_row,
      charge_embed, w0, wrest, w1b)


def _make_gather_kernel(n_edges, n_nodes):
    e_pw = n_edges // NW
    mesh = plsc.VectorSubcoreMesh(core_axis_name="c", subcore_axis_name="s")

    @functools.partial(
        pl.kernel, mesh=mesh,
        out_type=[jax.ShapeDtypeStruct((n_edges,), jnp.float32),
                  jax.ShapeDtypeStruct((n_edges,), jnp.int32)],
        compiler_params=pltpu.CompilerParams(needs_layout_passes=False),
        scratch_types=[
            pltpu.VMEM((n_nodes,), jnp.float32),
            pltpu.VMEM((n_nodes,), jnp.int32),
            pltpu.VMEM((e_pw,), jnp.int32),
            pltpu.VMEM((e_pw,), jnp.float32),
            pltpu.VMEM((e_pw,), jnp.int32),
        ],
    )
    def gather_kernel(ch_hbm, spn_hbm, snd_hbm, ceo_hbm, speo_hbm,
                      vch, vspn, vidx, voc, vos):
        wid = lax.axis_index("s") * 2 + lax.axis_index("c")
        base = wid * e_pw
        pltpu.sync_copy(ch_hbm, vch)
        pltpu.sync_copy(spn_hbm, vspn)
        pltpu.sync_copy(snd_hbm.at[pl.ds(base, e_pw)], vidx)

        def g_body(i, carry):
            sl = pl.ds(i * LANES, LANES)
            idx = vidx[sl]
            voc[sl] = plsc.load_gather(vch, [idx])
            vos[sl] = plsc.load_gather(vspn, [idx])
            return carry

        lax.fori_loop(0, e_pw // LANES, g_body, 0)

        pltpu.sync_copy(voc, ceo_hbm.at[pl.ds(base, e_pw)])
        pltpu.sync_copy(vos, speo_hbm.at[pl.ds(base, e_pw)])

    return gather_kernel


def _mlp_body(*refs):
    x_ref = refs[0]
    ce_refs = refs[1:1 + _NSEG]
    spe_refs = refs[1 + _NSEG:1 + 2 * _NSEG]
    env_refs = refs[1 + 2 * _NSEG:1 + 3 * _NSEG]
    w1a_ref, uvec_ref, sp_ref, w2_ref, w3_ref, out_ref = refs[1 + 3 * _NSEG:]
    n_sp = sp_ref.shape[0]
    b = x_ref.shape[0]
    c_row = jnp.concatenate([r[...].reshape(1, _SB) for r in ce_refs], axis=1)
    env_row = jnp.concatenate([r[...].reshape(1, _SB) for r in env_refs], axis=1)
    sp_row = jnp.concatenate([r[...].reshape(1, _SB) for r in spe_refs], axis=1)
    ioc = lax.broadcasted_iota(jnp.int32, (n_sp, 1), 0)
    oht = (ioc == sp_row).astype(jnp.bfloat16)
    rk1 = (((0,), (0,)), ((), ()))
    c_mat = lax.dot_general(c_row, uvec_ref[...], rk1,
                            preferred_element_type=jnp.float32)
    ones_row = jnp.ones((1, 128), jnp.float32)
    env_mat = lax.dot_general(env_row, ones_row, rk1,
                              preferred_element_type=jnp.float32)
    pre = (jnp.dot(x_ref[...].astype(jnp.bfloat16), w1a_ref[...],
                   preferred_element_type=jnp.float32)
           + c_mat
           + lax.dot_general(oht, sp_ref[...], rk1,
                             preferred_element_type=jnp.float32))
    h = _silu(pre)
    h = _silu(jnp.dot(h.astype(jnp.bfloat16), w2_ref[...],
                      preferred_element_type=jnp.float32))
    h = jnp.dot(h.astype(jnp.bfloat16), w3_ref[...],
                preferred_element_type=jnp.float32)
    out_ref[...] = env_mat * h


def _edge_stage2(x, c_e, sp_e, env_segs, w1a, uvec, sprime, W_x2, W_x3):
    n_edges = x.shape[0]
    n_sp = sprime.shape[0]
    grid = n_edges // _BE
    bps = (n_edges // _NSEG) // _SB
    blk = lambda shape: pl.BlockSpec(shape, lambda i: (i, 0))
    blk1 = pl.BlockSpec((_SB,), lambda i: (i,))
    alias_specs = [pl.BlockSpec((_SB,), (lambda j: (lambda i: (i + bps * j,)))(j))
                   for j in range(_NSEG)]
    full = lambda shape: pl.BlockSpec(shape, lambda i: (0, 0))
    return pl.pallas_call(
        _mlp_body,
        grid=(grid,),
        in_specs=[blk((_BE, 128))] + alias_specs + alias_specs
        + [blk1] * _NSEG + [
            full((128, 128)), full((1, 128)), full((n_sp, 128)),
            full((128, 128)), full((128, 128))],
        out_specs=blk((_BE, 128)),
        out_shape=jax.ShapeDtypeStruct((n_edges, 128), jnp.float32),
    )(x, *([c_e] * _NSEG), *([sp_e] * _NSEG), *env_segs,
      w1a, uvec, sprime, W_x2, W_x3)


def kernel(vectors, x, W_chi1, W_chi2, W_sig1, W_sig2, W_eps1, W_eps2,
           radius, hardness, charge_embed, W_w, W_x1, W_x2, W_x3,
           senders, species):
    n_edges = x.shape[0]
    n_nodes = species.shape[0]
    n_sp = radius.shape[0]

    wcat = jnp.concatenate([W_chi1, W_sig1, W_eps1], axis=1)
    w2bd = jnp.zeros((80, 8), jnp.float32)
    w2bd = w2bd.at[0:16, 0].set(W_chi2[:, 0])
    w2bd = w2bd.at[16:48, 1].set(W_sig2[:, 0])
    w2bd = w2bd.at[48:80, 2].set(W_eps2[:, 0])
    v3t = jnp.zeros((8, 3), jnp.float32).at[3, :].set(1.0)

    outs = _edge_stage1(
        vectors, x, wcat.astype(jnp.bfloat16),
        w2bd.T.astype(jnp.bfloat16), v3t)
    chis_segs = outs[0:_NSEG]
    sig_segs = outs[_NSEG:2 * _NSEG]
    eps_segs = outs[2 * _NSEG:3 * _NSEG]
    env_segs = outs[3 * _NSEG:4 * _NSEG]

    sndp = (senders.reshape(n_edges // _BE, _NSEG, _SB)
            .swapaxes(0, 1).reshape(n_edges))

    seg = _make_seg_kernel(n_edges, n_nodes)
    cparts, sparts, eparts = seg(
        jnp.concatenate(chis_segs), jnp.concatenate(sig_segs),
        jnp.concatenate(eps_segs), sndp)

    charges_row, pot11, vdw11, sprime, uvec = _node_stage(
        cparts, sparts, eparts,
        species.reshape(1, n_nodes),
        radius.reshape(1, n_sp), hardness.reshape(1, n_sp),
        charge_embed, W_w[0:1, :], W_w[1:, :], W_x1[128:, :])

    gat = _make_gather_kernel(n_edges, n_nodes)
    c_e, sp_e = gat(charges_row.reshape(n_nodes), species, sndp)

    x_out = _edge_stage2(x, c_e, sp_e, env_segs,
                         W_x1[:128, :].astype(jnp.bfloat16), uvec,
                         sprime.astype(jnp.bfloat16),
                         W_x2.astype(jnp.bfloat16), W_x3.astype(jnp.bfloat16))

    return (x_out, charges_row.reshape(n_nodes), pot11[0, 0], vdw11[0, 0])


# speedup vs baseline: 1.1706x; 1.1706x over previous
import functools
import math

import jax
import jax.numpy as jnp
from jax import lax
from jax.experimental import pallas as pl
from jax.experimental.pallas import tpu as pltpu
from jax.experimental.pallas import tpu_sc as plsc

N_NODES_C = 10000
NUM_SPECIES_C = 100
NW = 32
LANES = 16

_SB = 512
_NSEG = 25
_BE = _SB * _NSEG


def _silu(v):
    t = 0.5 * v
    return t * jnp.tanh(t) + t


def _edge1_body(vec_ref, x_ref, wcat_ref, w2bdt_ref, v3t_ref, *out_refs):
    xb16 = x_ref[...].astype(jnp.bfloat16)
    h = _silu(jnp.dot(xb16, wcat_ref[...], preferred_element_type=jnp.float32))
    v = vec_ref[...]
    tr = (((1,), (1,)), ((), ()))
    st = (lax.dot_general(w2bdt_ref[...], h.astype(jnp.bfloat16), tr,
                          preferred_element_type=jnp.float32)
          + lax.dot_general(v3t_ref[...], v * v, tr,
                            preferred_element_type=jnp.float32))
    r2 = st[3]
    u = jnp.clip(jnp.sqrt(r2), 0.0, 1.0)
    u2 = u * u
    u6 = u2 * u2 * u2
    env = 1.0 - 28.0 * u6 + 48.0 * u6 * u - 21.0 * u6 * u2
    rows = (st[0], st[1], st[2], env)
    for q in range(4):
        for j in range(_NSEG):
            out_refs[q * _NSEG + j][...] = rows[q][j * _SB:(j + 1) * _SB]


def _edge_stage1(vectors, x, wcat, w2bd, v3):
    n_edges = x.shape[0]
    grid = n_edges // _BE
    seg = n_edges // _NSEG
    blk = lambda shape: pl.BlockSpec(shape, lambda i: (i, 0))
    blk1 = pl.BlockSpec((_SB,), lambda i: (i,))
    full = lambda shape: pl.BlockSpec(shape, lambda i: (0, 0))
    out_sds = jax.ShapeDtypeStruct((seg,), jnp.float32)
    return pl.pallas_call(
        _edge1_body,
        grid=(grid,),
        in_specs=[blk((_BE, 3)), blk((_BE, 128)),
                  full((128, 80)), full((8, 80)), full((8, 3))],
        out_specs=[blk1] * (4 * _NSEG),
        out_shape=[out_sds] * (4 * _NSEG),
    )(vectors, x, wcat, w2bd, v3)


def _make_seg_kernel(n_edges, n_nodes):
    e_pw = n_edges // NW
    mesh = plsc.VectorSubcoreMesh(core_axis_name="c", subcore_axis_name="s")
    out_sds = jax.ShapeDtypeStruct((NW, n_nodes), jnp.float32)

    @functools.partial(
        pl.kernel, mesh=mesh,
        out_type=[out_sds] * 3,
        compiler_params=pltpu.CompilerParams(needs_layout_passes=False),
        scratch_types=[
            pltpu.VMEM((e_pw,), jnp.float32),
            pltpu.VMEM((e_pw,), jnp.float32),
            pltpu.VMEM((e_pw,), jnp.float32),
            pltpu.VMEM((e_pw,), jnp.int32),
            pltpu.VMEM((n_nodes,), jnp.float32),
            pltpu.VMEM((n_nodes,), jnp.float32),
            pltpu.VMEM((n_nodes,), jnp.float32),
        ],
    )
    def seg_kernel(ce_hbm, se_hbm, ee_hbm, snd_hbm,
                   outc_hbm, outs_hbm, oute_hbm,
                   vc, vs, ve, vidx, ac, a_s, ae):
        wid = lax.axis_index("s") * 2 + lax.axis_index("c")
        base = wid * e_pw
        pltpu.sync_copy(ce_hbm.at[pl.ds(base, e_pw)], vc)
        pltpu.sync_copy(se_hbm.at[pl.ds(base, e_pw)], vs)
        pltpu.sync_copy(ee_hbm.at[pl.ds(base, e_pw)], ve)
        pltpu.sync_copy(snd_hbm.at[pl.ds(base, e_pw)], vidx)

        zero = jnp.zeros((LANES,), jnp.float32)

        def zero_body(i, carry):
            sl = pl.ds(i * LANES, LANES)
            ac[sl] = zero
            a_s[sl] = zero
            ae[sl] = zero
            return carry

        lax.fori_loop(0, n_nodes // LANES, zero_body, 0)

        def scat_body(i, carry):
            sl = pl.ds(i * LANES, LANES)
            idx = vidx[sl]
            plsc.addupdate_scatter(ac, [idx], vc[sl])
            plsc.addupdate_scatter(a_s, [idx], vs[sl])
            plsc.addupdate_scatter(ae, [idx], ve[sl])
            return carry

        lax.fori_loop(0, e_pw // LANES, scat_body, 0)

        pltpu.sync_copy(ac, outc_hbm.at[wid])
        pltpu.sync_copy(a_s, outs_hbm.at[wid])
        pltpu.sync_copy(ae, oute_hbm.at[wid])

    return seg_kernel


def _node_body(cp_ref, sp_ref, ep_ref, spe_ref, rad_ref, hardn_ref,
               cemb_ref, w0_ref, wrest_ref, w1b_ref,
               charges_ref, pot_ref, vdw_ref, sprime_ref, uvec_ref):
    chis = jnp.sum(cp_ref[...], axis=0, keepdims=True)
    ssum = jnp.sum(sp_ref[...], axis=0, keepdims=True)
    esum = jnp.sum(ep_ref[...], axis=0, keepdims=True)
    spe = spe_ref[...]
    n_sp = rad_ref.shape[1]
    ioc = lax.broadcasted_iota(jnp.int32, (n_sp, 1), 0)
    oh = (spe == ioc).astype(jnp.float32)
    gam = jnp.dot(rad_ref[...], oh, preferred_element_type=jnp.float32) * 4.0 + 0.5
    hraw = jnp.dot(hardn_ref[...], oh, preferred_element_type=jnp.float32)
    hard = jnp.maximum(hraw, 0.0) + jnp.log1p(jnp.exp(-jnp.abs(hraw)))
    charges = -chis / hard
    pot_terms = (chis * charges + 0.5 * hard * charges * charges
                 + charges * charges / (gam * math.sqrt(math.pi)))
    pot_ref[...] = jnp.sum(pot_terms, keepdims=True)
    sigma = jax.nn.sigmoid(ssum) * 0.15 + 0.15
    epsn = jax.nn.sigmoid(esum) * 1.7 + 0.3
    s2 = sigma * sigma
    vdw_ref[...] = jnp.sum(epsn * s2 * s2 * s2, keepdims=True)
    charges_ref[...] = charges
    s_tab = jnp.dot(cemb_ref[...], wrest_ref[...], preferred_element_type=jnp.float32)
    sprime_ref[...] = jnp.dot(s_tab, w1b_ref[...], preferred_element_type=jnp.float32)
    uvec_ref[...] = jnp.dot(w0_ref[...], w1b_ref[...], preferred_element_type=jnp.float32)


def _node_stage(cparts, sparts, eparts, species_row, rad_row, hardn_row,
                charge_embed, w0, wrest, w1b):
    n_nodes = species_row.shape[1]
    n_sp = rad_row.shape[1]
    full = lambda shape: pl.BlockSpec(shape, lambda: (0, 0))
    return pl.pallas_call(
        _node_body,
        in_specs=[full((NW, n_nodes))] * 3 + [
            full((1, n_nodes)), full((1, n_sp)), full((1, n_sp)),
            full((n_sp, 16)), full((1, 16)), full((16, 16)), full((16, 128))],
        out_specs=[full((1, n_nodes)), full((1, 1)), full((1, 1)),
                   full((n_sp, 128)), full((1, 128))],
        out_shape=[jax.ShapeDtypeStruct((1, n_nodes), jnp.float32),
                   jax.ShapeDtypeStruct((1, 1), jnp.float32),
                   jax.ShapeDtypeStruct((1, 1), jnp.float32),
                   jax.ShapeDtypeStruct((n_sp, 128), jnp.float32),
                   jax.ShapeDtypeStruct((1, 128), jnp.float32)],
    )(cparts, sparts, eparts, species_row, rad_row, hardn_row,
      charge_embed, w0, wrest, w1b)


def _make_gather_kernel(n_edges, n_nodes):
    e_pw = n_edges // NW
    mesh = plsc.VectorSubcoreMesh(core_axis_name="c", subcore_axis_name="s")

    @functools.partial(
        pl.kernel, mesh=mesh,
        out_type=[jax.ShapeDtypeStruct((n_edges,), jnp.float32),
                  jax.ShapeDtypeStruct((n_edges,), jnp.int32)],
        compiler_params=pltpu.CompilerParams(needs_layout_passes=False),
        scratch_types=[
            pltpu.VMEM((n_nodes,), jnp.float32),
            pltpu.VMEM((n_nodes,), jnp.int32),
            pltpu.VMEM((e_pw,), jnp.int32),
            pltpu.VMEM((e_pw,), jnp.float32),
            pltpu.VMEM((e_pw,), jnp.int32),
        ],
    )
    def gather_kernel(ch_hbm, spn_hbm, snd_hbm, ceo_hbm, speo_hbm,
                      vch, vspn, vidx, voc, vos):
        wid = lax.axis_index("s") * 2 + lax.axis_index("c")
        base = wid * e_pw
        pltpu.sync_copy(ch_hbm, vch)
        pltpu.sync_copy(spn_hbm, vspn)
        pltpu.sync_copy(snd_hbm.at[pl.ds(base, e_pw)], vidx)

        def g_body(i, carry):
            sl = pl.ds(i * LANES, LANES)
            idx = vidx[sl]
            voc[sl] = plsc.load_gather(vch, [idx])
            vos[sl] = plsc.load_gather(vspn, [idx])
            return carry

        lax.fori_loop(0, e_pw // LANES, g_body, 0)

        pltpu.sync_copy(voc, ceo_hbm.at[pl.ds(base, e_pw)])
        pltpu.sync_copy(vos, speo_hbm.at[pl.ds(base, e_pw)])

    return gather_kernel


def _mlp_body(*refs):
    x_ref = refs[0]
    ce_refs = refs[1:1 + _NSEG]
    spe_refs = refs[1 + _NSEG:1 + 2 * _NSEG]
    env_refs = refs[1 + 2 * _NSEG:1 + 3 * _NSEG]
    w1a_ref, uvec_ref, sp_ref, w2_ref, w3_ref, out_ref = refs[1 + 3 * _NSEG:]
    n_sp = sp_ref.shape[0]
    b = x_ref.shape[0]
    c_row = jnp.concatenate([r[...].reshape(1, _SB) for r in ce_refs], axis=1)
    env_row = jnp.concatenate([r[...].reshape(1, _SB) for r in env_refs], axis=1)
    sp_row = jnp.concatenate([r[...].reshape(1, _SB) for r in spe_refs], axis=1)
    ioc = lax.broadcasted_iota(jnp.int32, (n_sp, 1), 0)
    oht = (ioc == sp_row).astype(jnp.bfloat16)
    rk1 = (((0,), (0,)), ((), ()))
    c_mat = lax.dot_general(c_row, uvec_ref[...], rk1,
                            preferred_element_type=jnp.float32)
    ones_row = jnp.ones((1, 128), jnp.float32)
    env_mat = lax.dot_general(env_row, ones_row, rk1,
                              preferred_element_type=jnp.float32)
    pre = (jnp.dot(x_ref[...].astype(jnp.bfloat16), w1a_ref[...],
                   preferred_element_type=jnp.float32)
           + c_mat
           + lax.dot_general(oht, sp_ref[...], rk1,
                             preferred_element_type=jnp.float32))
    h = _silu(pre)
    h = _silu(jnp.dot(h.astype(jnp.bfloat16), w2_ref[...],
                      preferred_element_type=jnp.float32))
    h = jnp.dot(h.astype(jnp.bfloat16), w3_ref[...],
                preferred_element_type=jnp.float32)
    out_ref[...] = env_mat * h


def _edge_stage2(x, c_e, sp_e, env_segs, w1a, uvec, sprime, W_x2, W_x3):
    n_edges = x.shape[0]
    n_sp = sprime.shape[0]
    grid = n_edges // _BE
    bps = (n_edges // _NSEG) // _SB
    blk = lambda shape: pl.BlockSpec(shape, lambda i: (i, 0))
    blk1 = pl.BlockSpec((_SB,), lambda i: (i,))
    alias_specs = [pl.BlockSpec((_SB,), (lambda j: (lambda i: (i + bps * j,)))(j))
                   for j in range(_NSEG)]
    full = lambda shape: pl.BlockSpec(shape, lambda i: (0, 0))
    return pl.pallas_call(
        _mlp_body,
        grid=(grid,),
        in_specs=[blk((_BE, 128))] + alias_specs + alias_specs
        + [blk1] * _NSEG + [
            full((128, 128)), full((1, 128)), full((n_sp, 128)),
            full((128, 128)), full((128, 128))],
        out_specs=blk((_BE, 128)),
        out_shape=jax.ShapeDtypeStruct((n_edges, 128), jnp.float32),
    )(x, *([c_e] * _NSEG), *([sp_e] * _NSEG), *env_segs,
      w1a, uvec, sprime, W_x2, W_x3)


def kernel(vectors, x, W_chi1, W_chi2, W_sig1, W_sig2, W_eps1, W_eps2,
           radius, hardness, charge_embed, W_w, W_x1, W_x2, W_x3,
           senders, species):
    n_edges = x.shape[0]
    n_nodes = species.shape[0]
    n_sp = radius.shape[0]

    wcat = jnp.concatenate([W_chi1, W_sig1, W_eps1], axis=1)
    w2bd = jnp.zeros((80, 8), jnp.float32)
    w2bd = w2bd.at[0:16, 0].set(W_chi2[:, 0])
    w2bd = w2bd.at[16:48, 1].set(W_sig2[:, 0])
    w2bd = w2bd.at[48:80, 2].set(W_eps2[:, 0])
    v3t = jnp.zeros((8, 3), jnp.float32).at[3, :].set(1.0)

    outs = _edge_stage1(
        vectors, x, wcat.astype(jnp.bfloat16),
        w2bd.T.astype(jnp.bfloat16), v3t)
    chis_segs = outs[0:_NSEG]
    sig_segs = outs[_NSEG:2 * _NSEG]
    eps_segs = outs[2 * _NSEG:3 * _NSEG]
    env_segs = outs[3 * _NSEG:4 * _NSEG]

    sndp = (senders.reshape(n_edges // _BE, _NSEG, _SB)
            .swapaxes(0, 1).reshape(n_edges))

    seg = _make_seg_kernel(n_edges, n_nodes)
    cparts, sparts, eparts = seg(
        jnp.concatenate(chis_segs), jnp.concatenate(sig_segs),
        jnp.concatenate(eps_segs), sndp)

    charges_row, pot11, vdw11, sprime, uvec = _node_stage(
        cparts, sparts, eparts,
        species.reshape(1, n_nodes),
        radius.reshape(1, n_sp), hardness.reshape(1, n_sp),
        charge_embed, W_w[0:1, :], W_w[1:, :], W_x1[128:, :])

    gat = _make_gather_kernel(n_edges, n_nodes)
    c_e, sp_e = gat(charges_row.reshape(n_nodes), species, sndp)

    x_out = _edge_stage2(x, c_e, sp_e, env_segs,
                         W_x1[:128, :].astype(jnp.bfloat16), uvec,
                         sprime.astype(jnp.bfloat16),
                         W_x2.astype(jnp.bfloat16), W_x3.astype(jnp.bfloat16))

    return (x_out, charges_row.reshape(n_nodes), pot11[0, 0], vdw11[0, 0])
